# baseline (speedup 1.0000x reference)
import functools

import jax
import jax.numpy as jnp
import numpy as np
from jax import lax
from jax.experimental import pallas as pl
from jax.experimental.pallas import tpu as pltpu
from jax.experimental.pallas import tpu_sc as plsc

N_EMB = 8192
EMB_DIM = 32
BETA = 0.25
THRESHOLD = 1.0
ROWS = 16384
T = 256
NT = ROWS // T
AUG = 128
NW = 32
RPW = ROWS // NW
PPW = N_EMB // NW

try:
    _PERM = np.asarray(jax.random.permutation(jax.random.key(42), ROWS))
except Exception:
    _PERM = None


def _perm_idx():
    if _PERM is not None:
        return _PERM[:N_EMB]
    return jax.random.permutation(jax.random.key(42), ROWS)[:N_EMB]


def _k1(xaug_ref, embM_ref, idx_ref, stats_ref):
    i = pl.program_id(0)
    rows_aug = xaug_ref[...]
    rows = rows_aug[:, :EMB_DIM]
    sim2 = jnp.dot(rows, embM_ref[:EMB_DIM, :],
                   preferred_element_type=jnp.float32)
    val = embM_ref[EMB_DIM:EMB_DIM + 1, :] + sim2
    minv = jnp.min(val, axis=1, keepdims=True)
    idx = jnp.argmin(val, axis=1).astype(jnp.int32)
    idx_ref[0, 0, :] = idx
    s1 = jnp.sum(rows * rows, axis=1, keepdims=True)

    @pl.when(i == 0)
    def _():
        stats_ref[0] = 0.0
        stats_ref[1] = 0.0
        stats_ref[2] = 0.0

    stats_ref[0] += jnp.sum(rows)
    stats_ref[1] += jnp.sum(s1)
    stats_ref[2] += jnp.sum(jnp.nan_to_num(s1 + minv))


def _sc_seg_body(flat_hbm, idx_hbm, perm_hbm, zeros_hbm, seg_hbm, krand_hbm,
                 idxv, rowsv, pidxv, prowv, table_sh, sem):
    c = lax.axis_index("c")
    s = lax.axis_index("s")
    w = s * 2 + c
    pltpu.sync_copy(zeros_hbm, table_sh.at[pl.ds(RPW * s, RPW)])
    plsc.subcore_barrier()
    pltpu.sync_copy(idx_hbm.at[pl.ds(4 * w, 4)], idxv)
    for j in range(4):
        pltpu.sync_copy(flat_hbm.at[pl.ds(RPW * w + 128 * j, 128)], rowsv)
        pltpu.sync_copy(rowsv, table_sh.at[idxv.at[j]], add=True)
    plsc.subcore_barrier()
    pltpu.sync_copy(table_sh.at[pl.ds(RPW * s, RPW)],
                    seg_hbm.at[c, pl.ds(RPW * s, RPW)])
    pltpu.sync_copy(perm_hbm.at[pl.ds(2 * w, 2)], pidxv)
    for j in range(2):
        pltpu.async_copy(flat_hbm.at[pidxv.at[j]], prowv, sem).wait()
        pltpu.sync_copy(prowv, krand_hbm.at[pl.ds(PPW * w + 128 * j, 128)])


_sc_seg = functools.partial(
    pl.kernel,
    mesh=plsc.VectorSubcoreMesh(core_axis_name="c", subcore_axis_name="s"),
    out_type=[
        jax.ShapeDtypeStruct((2, N_EMB, AUG), jnp.float32),
        jax.ShapeDtypeStruct((N_EMB, AUG), jnp.float32),
    ],
    scratch_types=[
        pltpu.VMEM((4, 128), jnp.int32),
        pltpu.VMEM((128, AUG), jnp.float32),
        pltpu.VMEM((2, 128), jnp.int32),
        pltpu.VMEM((128, AUG), jnp.float32),
        pltpu.VMEM_SHARED((N_EMB, AUG), jnp.float32),
        pltpu.SemaphoreType.DMA,
    ],
)(_sc_seg_body)


def _k3(seg_ref, emb_ref, krand_ref, newk_ref, sc_ref, used_ref):
    seg = seg_ref[0] + seg_ref[1]
    ksum_new = seg[:, :EMB_DIM]
    kelem_new = seg[:, EMB_DIM:EMB_DIM + 1]
    emb = emb_ref[...]
    k_sum = BETA * emb + (1.0 - BETA) * ksum_new
    k_elem = BETA * 1.0 + (1.0 - BETA) * kelem_new
    usage = (k_elem >= THRESHOLD).astype(jnp.float32)
    new_k = usage * (k_sum / k_elem) + (1.0 - usage) * krand_ref[:, :EMB_DIM]
    newk_ref[...] = new_k
    prob = kelem_new / jnp.sum(kelem_new)
    sc_ref[0] = -jnp.sum(prob * jnp.log(prob + 1e-8))
    sc_ref[1] = jnp.sum(usage)
    diff = new_k - emb
    sc_ref[2] = jnp.sum(diff * diff)
    used_ref[0] = jnp.sum((kelem_new >= THRESHOLD).astype(jnp.int32))


def _k2(idx_ref, x_ref, newk_ref, q_ref, comm_ref):
    i = pl.program_id(0)
    idx = idx_ref[0, 0, :]
    rows = x_ref[...]
    newk = newk_ref[...]
    codes = jax.lax.broadcasted_iota(jnp.int32, (T, N_EMB), 1)
    one_hot = (codes == idx[:, None]).astype(jnp.bfloat16)
    q = jnp.dot(one_hot, newk.astype(jnp.bfloat16),
                preferred_element_type=jnp.float32)
    q_ref[...] = q

    @pl.when(i == 0)
    def _():
        comm_ref[0] = 0.0

    d = q - rows
    comm_ref[0] += jnp.sum(d * d)


def kernel(x, embeddings):
    xt = jnp.swapaxes(x, 1, -1)
    flat_x = xt.reshape(ROWS, EMB_DIM)
    xaug = jnp.pad(flat_x, ((0, 0), (0, AUG - EMB_DIM)))
    xaug = xaug.at[:, EMB_DIM].set(1.0)
    embT = embeddings.T
    s2 = jnp.sum(embT * embT, axis=0, keepdims=True)
    embM = jnp.concatenate(
        [-2.0 * embT, s2,
         jnp.zeros((AUG - EMB_DIM - 1, N_EMB), jnp.float32)], axis=0)

    idx3, stats = pl.pallas_call(
        _k1,
        grid=(NT,),
        in_specs=[
            pl.BlockSpec((T, AUG), lambda i: (i, 0)),
            pl.BlockSpec((AUG, N_EMB), lambda i: (0, 0)),
        ],
        out_specs=[
            pl.BlockSpec((1, 1, T), lambda i: (i, 0, 0)),
            pl.BlockSpec(memory_space=pltpu.SMEM),
        ],
        out_shape=[
            jax.ShapeDtypeStruct((NT, 1, T), jnp.int32),
            jax.ShapeDtypeStruct((4,), jnp.float32),
        ],
    )(xaug, embM)

    idx2d = idx3.reshape(128, 128)
    perm2d = jnp.asarray(_perm_idx(), jnp.int32).reshape(64, 128)
    zeros = jnp.zeros((RPW, AUG), jnp.float32)
    seg2, krand = _sc_seg(xaug, idx2d, perm2d, zeros)

    new_k, sc, usedc = pl.pallas_call(
        _k3,
        in_specs=[
            pl.BlockSpec((2, N_EMB, AUG), lambda: (0, 0, 0)),
            pl.BlockSpec((N_EMB, EMB_DIM), lambda: (0, 0)),
            pl.BlockSpec((N_EMB, AUG), lambda: (0, 0)),
        ],
        out_specs=[
            pl.BlockSpec((N_EMB, EMB_DIM), lambda: (0, 0)),
            pl.BlockSpec(memory_space=pltpu.SMEM),
            pl.BlockSpec(memory_space=pltpu.SMEM),
        ],
        out_shape=[
            jax.ShapeDtypeStruct((N_EMB, EMB_DIM), jnp.float32),
            jax.ShapeDtypeStruct((4,), jnp.float32),
            jax.ShapeDtypeStruct((1,), jnp.int32),
        ],
    )(seg2, embeddings, krand)

    q_flat, comm = pl.pallas_call(
        _k2,
        grid=(NT,),
        in_specs=[
            pl.BlockSpec((1, 1, T), lambda i: (i, 0, 0)),
            pl.BlockSpec((T, EMB_DIM), lambda i: (i, 0)),
            pl.BlockSpec((N_EMB, EMB_DIM), lambda i: (0, 0)),
        ],
        out_specs=[
            pl.BlockSpec((T, EMB_DIM), lambda i: (i, 0)),
            pl.BlockSpec(memory_space=pltpu.SMEM),
        ],
        out_shape=[
            jax.ShapeDtypeStruct((ROWS, EMB_DIM), jnp.float32),
            jax.ShapeDtypeStruct((1,), jnp.float32),
        ],
    )(idx3, flat_x, new_k)

    quantized = jnp.swapaxes(q_flat.reshape(xt.shape), 1, -1)
    out = quantized

    n = float(ROWS * EMB_DIM)
    mean = stats[0] / n
    prenorm = jnp.sqrt(jnp.maximum(stats[1] - n * mean * mean, 0.0) / n)
    fit = stats[2] / float(ROWS)
    loss = BETA * comm[0] / n
    entropy = sc[0]
    usage_sum = sc[1]
    dk = jnp.nan_to_num(jnp.sqrt(sc[2]) / np.sqrt(float(N_EMB * EMB_DIM)))
    used_curr = usedc[0]
    return (out, quantized, loss, fit, prenorm, entropy, used_curr,
            usage_sum, dk)


# speedup vs baseline: 1.0022x; 1.0022x over previous
import functools

import jax
import jax.numpy as jnp
import numpy as np
from jax import lax
from jax.experimental import pallas as pl
from jax.experimental.pallas import tpu as pltpu
from jax.experimental.pallas import tpu_sc as plsc

N_EMB = 8192
EMB_DIM = 32
BETA = 0.25
THRESHOLD = 1.0
ROWS = 16384
T = 256
NT = ROWS // T
AUG = 128
NW = 32
RPW = ROWS // NW
PPW = N_EMB // NW

try:
    _PERM = np.asarray(jax.random.permutation(jax.random.key(42), ROWS))
except Exception:
    _PERM = None


def _perm_idx():
    if _PERM is not None:
        return _PERM[:N_EMB]
    return jax.random.permutation(jax.random.key(42), ROWS)[:N_EMB]


def _k1(xaug_ref, embM_ref, idx_ref, stats_ref):
    i = pl.program_id(0)
    rows_aug = xaug_ref[...]
    rows = rows_aug[:, :EMB_DIM]
    sim2 = jnp.dot(rows, embM_ref[:EMB_DIM, :],
                   preferred_element_type=jnp.float32)
    val = embM_ref[EMB_DIM:EMB_DIM + 1, :] + sim2
    minv = jnp.min(val, axis=1, keepdims=True)
    idx = jnp.argmin(val, axis=1).astype(jnp.int32)
    idx_ref[0, 0, :] = idx
    s1 = jnp.sum(rows * rows, axis=1, keepdims=True)

    @pl.when(i == 0)
    def _():
        stats_ref[0] = 0.0
        stats_ref[1] = 0.0
        stats_ref[2] = 0.0

    stats_ref[0] += jnp.sum(rows)
    stats_ref[1] += jnp.sum(s1)
    stats_ref[2] += jnp.sum(jnp.nan_to_num(s1 + minv))


def _sc_seg_body(flat_hbm, idx_hbm, perm_hbm, zeros_hbm, seg_hbm, krand_hbm,
                 idxv, rowsv, pidxv, prowv, table_sh, sem):
    c = lax.axis_index("c")
    s = lax.axis_index("s")
    w = s * 2 + c
    pltpu.sync_copy(zeros_hbm, table_sh.at[pl.ds(RPW * s, RPW)])
    plsc.subcore_barrier()
    pltpu.sync_copy(idx_hbm.at[pl.ds(4 * w, 4)], idxv)
    for j in range(4):
        pltpu.sync_copy(flat_hbm.at[pl.ds(RPW * w + 128 * j, 128)], rowsv)
        pltpu.sync_copy(rowsv, table_sh.at[idxv.at[j]], add=True)
    plsc.subcore_barrier()
    pltpu.sync_copy(table_sh.at[pl.ds(RPW * s, RPW)],
                    seg_hbm.at[c, pl.ds(RPW * s, RPW)])
    pltpu.sync_copy(perm_hbm.at[pl.ds(2 * w, 2)], pidxv)
    for j in range(2):
        pltpu.async_copy(flat_hbm.at[pidxv.at[j]], prowv, sem).wait()
        pltpu.sync_copy(prowv, krand_hbm.at[pl.ds(PPW * w + 128 * j, 128)])


_sc_seg = functools.partial(
    pl.kernel,
    mesh=plsc.VectorSubcoreMesh(core_axis_name="c", subcore_axis_name="s"),
    out_type=[
        jax.ShapeDtypeStruct((2, N_EMB, AUG), jnp.float32),
        jax.ShapeDtypeStruct((N_EMB, AUG), jnp.float32),
    ],
    scratch_types=[
        pltpu.VMEM((4, 128), jnp.int32),
        pltpu.VMEM((128, AUG), jnp.float32),
        pltpu.VMEM((2, 128), jnp.int32),
        pltpu.VMEM((128, AUG), jnp.float32),
        pltpu.VMEM_SHARED((N_EMB, AUG), jnp.float32),
        pltpu.SemaphoreType.DMA,
    ],
)(_sc_seg_body)


def _k3(seg_ref, emb_ref, krand_ref, newk_ref, sc_ref, used_ref):
    seg = seg_ref[0] + seg_ref[1]
    ksum_new = seg[:, :EMB_DIM]
    kelem_new = seg[:, EMB_DIM:EMB_DIM + 1]
    emb = emb_ref[...]
    k_sum = BETA * emb + (1.0 - BETA) * ksum_new
    k_elem = BETA * 1.0 + (1.0 - BETA) * kelem_new
    usage = (k_elem >= THRESHOLD).astype(jnp.float32)
    new_k = usage * (k_sum / k_elem) + (1.0 - usage) * krand_ref[:, :EMB_DIM]
    newk_ref[...] = new_k
    prob = kelem_new / jnp.sum(kelem_new)
    sc_ref[0] = -jnp.sum(prob * jnp.log(prob + 1e-8))
    sc_ref[1] = jnp.sum(usage)
    diff = new_k - emb
    sc_ref[2] = jnp.sum(diff * diff)
    used_ref[0] = jnp.sum((kelem_new >= THRESHOLD).astype(jnp.int32))


def _k2(idx_ref, x_ref, newk_ref, q_ref, comm_ref):
    i = pl.program_id(0)
    idx = idx_ref[0, 0, :]
    rows = x_ref[...]
    newk = newk_ref[...]
    codes = jax.lax.broadcasted_iota(jnp.int32, (T, N_EMB), 1)
    one_hot = (codes == idx[:, None]).astype(jnp.float32)
    q = jnp.dot(one_hot, newk, preferred_element_type=jnp.float32)
    q_ref[...] = q

    @pl.when(i == 0)
    def _():
        comm_ref[0] = 0.0

    d = q - rows
    comm_ref[0] += jnp.sum(d * d)


def kernel(x, embeddings):
    xt = jnp.swapaxes(x, 1, -1)
    flat_x = xt.reshape(ROWS, EMB_DIM)
    xaug = jnp.pad(flat_x, ((0, 0), (0, AUG - EMB_DIM)))
    xaug = xaug.at[:, EMB_DIM].set(1.0)
    embT = embeddings.T
    s2 = jnp.sum(embT * embT, axis=0, keepdims=True)
    embM = jnp.concatenate(
        [-2.0 * embT, s2,
         jnp.zeros((AUG - EMB_DIM - 1, N_EMB), jnp.float32)], axis=0)

    idx3, stats = pl.pallas_call(
        _k1,
        grid=(NT,),
        in_specs=[
            pl.BlockSpec((T, AUG), lambda i: (i, 0)),
            pl.BlockSpec((AUG, N_EMB), lambda i: (0, 0)),
        ],
        out_specs=[
            pl.BlockSpec((1, 1, T), lambda i: (i, 0, 0)),
            pl.BlockSpec(memory_space=pltpu.SMEM),
        ],
        out_shape=[
            jax.ShapeDtypeStruct((NT, 1, T), jnp.int32),
            jax.ShapeDtypeStruct((4,), jnp.float32),
        ],
    )(xaug, embM)

    idx2d = idx3.reshape(128, 128)
    perm2d = jnp.asarray(_perm_idx(), jnp.int32).reshape(64, 128)
    zeros = jnp.zeros((RPW, AUG), jnp.float32)
    seg2, krand = _sc_seg(xaug, idx2d, perm2d, zeros)

    new_k, sc, usedc = pl.pallas_call(
        _k3,
        in_specs=[
            pl.BlockSpec((2, N_EMB, AUG), lambda: (0, 0, 0)),
            pl.BlockSpec((N_EMB, EMB_DIM), lambda: (0, 0)),
            pl.BlockSpec((N_EMB, AUG), lambda: (0, 0)),
        ],
        out_specs=[
            pl.BlockSpec((N_EMB, EMB_DIM), lambda: (0, 0)),
            pl.BlockSpec(memory_space=pltpu.SMEM),
            pl.BlockSpec(memory_space=pltpu.SMEM),
        ],
        out_shape=[
            jax.ShapeDtypeStruct((N_EMB, EMB_DIM), jnp.float32),
            jax.ShapeDtypeStruct((4,), jnp.float32),
            jax.ShapeDtypeStruct((1,), jnp.int32),
        ],
    )(seg2, embeddings, krand)

    q_flat, comm = pl.pallas_call(
        _k2,
        grid=(NT,),
        in_specs=[
            pl.BlockSpec((1, 1, T), lambda i: (i, 0, 0)),
            pl.BlockSpec((T, EMB_DIM), lambda i: (i, 0)),
            pl.BlockSpec((N_EMB, EMB_DIM), lambda i: (0, 0)),
        ],
        out_specs=[
            pl.BlockSpec((T, EMB_DIM), lambda i: (i, 0)),
            pl.BlockSpec(memory_space=pltpu.SMEM),
        ],
        out_shape=[
            jax.ShapeDtypeStruct((ROWS, EMB_DIM), jnp.float32),
            jax.ShapeDtypeStruct((1,), jnp.float32),
        ],
    )(idx3, flat_x, new_k)

    quantized = jnp.swapaxes(q_flat.reshape(xt.shape), 1, -1)
    out = quantized

    n = float(ROWS * EMB_DIM)
    mean = stats[0] / n
    prenorm = jnp.sqrt(jnp.maximum(stats[1] - n * mean * mean, 0.0) / n)
    fit = stats[2] / float(ROWS)
    loss = BETA * comm[0] / n
    entropy = sc[0]
    usage_sum = sc[1]
    dk = jnp.nan_to_num(jnp.sqrt(sc[2]) / np.sqrt(float(N_EMB * EMB_DIM)))
    used_curr = usedc[0]
    return (out, quantized, loss, fit, prenorm, entropy, used_curr,
            usage_sum, dk)


# speedup vs baseline: 1.0065x; 1.0043x over previous
import functools

import jax
import jax.numpy as jnp
import numpy as np
from jax import lax
from jax.experimental import pallas as pl
from jax.experimental.pallas import tpu as pltpu
from jax.experimental.pallas import tpu_sc as plsc

N_EMB = 8192
EMB_DIM = 32
BETA = 0.25
THRESHOLD = 1.0
ROWS = 16384
T = 256
NT = ROWS // T
AUG = 128
NW = 32
RPW = ROWS // NW
PPW = N_EMB // NW

try:
    _PERM = np.asarray(jax.random.permutation(jax.random.key(42), ROWS))
except Exception:
    _PERM = None


def _perm_idx():
    if _PERM is not None:
        return _PERM[:N_EMB]
    return jax.random.permutation(jax.random.key(42), ROWS)[:N_EMB]


def _k1(xaug_ref, embM_ref, idx_ref, stats_ref):
    i = pl.program_id(0)
    rows_aug = xaug_ref[...]
    rows = rows_aug[:, :EMB_DIM]
    sim2 = jnp.dot(rows, embM_ref[:EMB_DIM, :],
                   preferred_element_type=jnp.float32)
    val = embM_ref[EMB_DIM:EMB_DIM + 1, :] + sim2
    minv = jnp.min(val, axis=1, keepdims=True)
    idx = jnp.argmin(val, axis=1).astype(jnp.int32)
    idx_ref[0, 0, :] = idx
    s1 = jnp.sum(rows * rows, axis=1, keepdims=True)

    @pl.when(i == 0)
    def _():
        stats_ref[0] = 0.0
        stats_ref[1] = 0.0
        stats_ref[2] = 0.0

    stats_ref[0] += jnp.sum(rows)
    stats_ref[1] += jnp.sum(s1)
    stats_ref[2] += jnp.sum(jnp.nan_to_num(s1 + minv))


def _sc_seg_body(flat_hbm, idx_hbm, perm_hbm, zeros_hbm, seg_hbm, krand_hbm,
                 idxv, rowsv, pidxv, prowv, table_sh, sem):
    c = lax.axis_index("c")
    s = lax.axis_index("s")
    w = s * 2 + c
    pltpu.sync_copy(zeros_hbm, table_sh.at[pl.ds(RPW * s, RPW)])
    plsc.subcore_barrier()
    pltpu.sync_copy(idx_hbm.at[pl.ds(4 * w, 4)], idxv)
    for j in range(4):
        pltpu.sync_copy(flat_hbm.at[pl.ds(RPW * w + 128 * j, 128)], rowsv)
        pltpu.sync_copy(rowsv, table_sh.at[idxv.at[j]], add=True)
    plsc.subcore_barrier()
    pltpu.sync_copy(table_sh.at[pl.ds(RPW * s, RPW)],
                    seg_hbm.at[c, pl.ds(RPW * s, RPW)])
    pltpu.sync_copy(perm_hbm.at[pl.ds(2 * w, 2)], pidxv)
    for j in range(2):
        pltpu.async_copy(flat_hbm.at[pidxv.at[j]], prowv, sem).wait()
        pltpu.sync_copy(prowv, krand_hbm.at[pl.ds(PPW * w + 128 * j, 128)])


_sc_seg = functools.partial(
    pl.kernel,
    mesh=plsc.VectorSubcoreMesh(core_axis_name="c", subcore_axis_name="s"),
    out_type=[
        jax.ShapeDtypeStruct((2, N_EMB, AUG), jnp.float32),
        jax.ShapeDtypeStruct((N_EMB, AUG), jnp.float32),
    ],
    scratch_types=[
        pltpu.VMEM((4, 128), jnp.int32),
        pltpu.VMEM((128, AUG), jnp.float32),
        pltpu.VMEM((2, 128), jnp.int32),
        pltpu.VMEM((128, AUG), jnp.float32),
        pltpu.VMEM_SHARED((N_EMB, AUG), jnp.float32),
        pltpu.SemaphoreType.DMA,
    ],
)(_sc_seg_body)


def _k23(idx_ref, x_ref, seg_ref, emb_ref, krand_ref,
         q_ref, sc_ref, used_ref, comm_ref, newk_scr):
    i = pl.program_id(0)

    @pl.when(i == 0)
    def _():
        seg = seg_ref[0] + seg_ref[1]
        ksum_new = seg[:, :EMB_DIM]
        kelem_new = seg[:, EMB_DIM:EMB_DIM + 1]
        emb = emb_ref[...]
        k_sum = BETA * emb + (1.0 - BETA) * ksum_new
        k_elem = BETA * 1.0 + (1.0 - BETA) * kelem_new
        usage = (k_elem >= THRESHOLD).astype(jnp.float32)
        new_k = (usage * (k_sum / k_elem)
                 + (1.0 - usage) * krand_ref[:, :EMB_DIM])
        newk_scr[...] = new_k
        prob = kelem_new / jnp.sum(kelem_new)
        sc_ref[0] = -jnp.sum(prob * jnp.log(prob + 1e-8))
        sc_ref[1] = jnp.sum(usage)
        diff = new_k - emb
        sc_ref[2] = jnp.sum(diff * diff)
        used_ref[0] = jnp.sum((kelem_new >= THRESHOLD).astype(jnp.int32))
        comm_ref[0] = 0.0

    idx = idx_ref[0, 0, :]
    rows = x_ref[...]
    newk = newk_scr[...]
    codes = jax.lax.broadcasted_iota(jnp.int32, (T, N_EMB), 1)
    one_hot = (codes == idx[:, None]).astype(jnp.float32)
    q = jnp.dot(one_hot, newk, preferred_element_type=jnp.float32)
    q_ref[...] = q
    d = q - rows
    comm_ref[0] += jnp.sum(d * d)


def kernel(x, embeddings):
    xt = jnp.swapaxes(x, 1, -1)
    flat_x = xt.reshape(ROWS, EMB_DIM)
    xaug = jnp.pad(flat_x, ((0, 0), (0, AUG - EMB_DIM)))
    xaug = xaug.at[:, EMB_DIM].set(1.0)
    embT = embeddings.T
    s2 = jnp.sum(embT * embT, axis=0, keepdims=True)
    embM = jnp.concatenate(
        [-2.0 * embT, s2,
         jnp.zeros((AUG - EMB_DIM - 1, N_EMB), jnp.float32)], axis=0)

    idx3, stats = pl.pallas_call(
        _k1,
        grid=(NT,),
        in_specs=[
            pl.BlockSpec((T, AUG), lambda i: (i, 0)),
            pl.BlockSpec((AUG, N_EMB), lambda i: (0, 0)),
        ],
        out_specs=[
            pl.BlockSpec((1, 1, T), lambda i: (i, 0, 0)),
            pl.BlockSpec(memory_space=pltpu.SMEM),
        ],
        out_shape=[
            jax.ShapeDtypeStruct((NT, 1, T), jnp.int32),
            jax.ShapeDtypeStruct((4,), jnp.float32),
        ],
    )(xaug, embM)

    idx2d = idx3.reshape(128, 128)
    perm2d = jnp.asarray(_perm_idx(), jnp.int32).reshape(64, 128)
    zeros = jnp.zeros((RPW, AUG), jnp.float32)
    seg2, krand = _sc_seg(xaug, idx2d, perm2d, zeros)

    q_flat, sc, usedc, comm = pl.pallas_call(
        _k23,
        grid=(NT,),
        in_specs=[
            pl.BlockSpec((1, 1, T), lambda i: (i, 0, 0)),
            pl.BlockSpec((T, EMB_DIM), lambda i: (i, 0)),
            pl.BlockSpec((2, N_EMB, AUG), lambda i: (0, 0, 0)),
            pl.BlockSpec((N_EMB, EMB_DIM), lambda i: (0, 0)),
            pl.BlockSpec((N_EMB, AUG), lambda i: (0, 0)),
        ],
        out_specs=[
            pl.BlockSpec((T, EMB_DIM), lambda i: (i, 0)),
            pl.BlockSpec(memory_space=pltpu.SMEM),
            pl.BlockSpec(memory_space=pltpu.SMEM),
            pl.BlockSpec(memory_space=pltpu.SMEM),
        ],
        out_shape=[
            jax.ShapeDtypeStruct((ROWS, EMB_DIM), jnp.float32),
            jax.ShapeDtypeStruct((4,), jnp.float32),
            jax.ShapeDtypeStruct((1,), jnp.int32),
            jax.ShapeDtypeStruct((1,), jnp.float32),
        ],
        scratch_shapes=[pltpu.VMEM((N_EMB, EMB_DIM), jnp.float32)],
    )(idx3, flat_x, seg2, embeddings, krand)

    quantized = jnp.swapaxes(q_flat.reshape(xt.shape), 1, -1)
    out = quantized

    n = float(ROWS * EMB_DIM)
    mean = stats[0] / n
    prenorm = jnp.sqrt(jnp.maximum(stats[1] - n * mean * mean, 0.0) / n)
    fit = stats[2] / float(ROWS)
    loss = BETA * comm[0] / n
    entropy = sc[0]
    usage_sum = sc[1]
    dk = jnp.nan_to_num(jnp.sqrt(sc[2]) / np.sqrt(float(N_EMB * EMB_DIM)))
    used_curr = usedc[0]
    return (out, quantized, loss, fit, prenorm, entropy, used_curr,
            usage_sum, dk)


# speedup vs baseline: 1.0501x; 1.0433x over previous
import functools

import jax
import jax.numpy as jnp
import numpy as np
from jax import lax
from jax.experimental import pallas as pl
from jax.experimental.pallas import tpu as pltpu
from jax.experimental.pallas import tpu_sc as plsc

N_EMB = 8192
EMB_DIM = 32
BETA = 0.25
THRESHOLD = 1.0
ROWS = 16384
T = 512
NT = ROWS // T
AUG = 128
NW = 32
RPW = ROWS // NW
PPW = N_EMB // NW

try:
    _PERM = np.asarray(jax.random.permutation(jax.random.key(42), ROWS))
except Exception:
    _PERM = None


def _perm_idx():
    if _PERM is not None:
        return _PERM[:N_EMB]
    return jax.random.permutation(jax.random.key(42), ROWS)[:N_EMB]


def _k1(xaug_ref, embM_ref, idx_ref, stats_ref):
    i = pl.program_id(0)
    rows_aug = xaug_ref[...]
    rows = rows_aug[:, :EMB_DIM]
    sim2 = jnp.dot(rows, embM_ref[:EMB_DIM, :],
                   preferred_element_type=jnp.float32)
    val = embM_ref[EMB_DIM:EMB_DIM + 1, :] + sim2
    minv = jnp.min(val, axis=1, keepdims=True)
    idx = jnp.argmin(val, axis=1).astype(jnp.int32)
    idx_ref[0, 0, :] = idx
    s1 = jnp.sum(rows * rows, axis=1, keepdims=True)

    @pl.when(i == 0)
    def _():
        stats_ref[0] = 0.0
        stats_ref[1] = 0.0
        stats_ref[2] = 0.0

    stats_ref[0] += jnp.sum(rows)
    stats_ref[1] += jnp.sum(s1)
    stats_ref[2] += jnp.sum(jnp.nan_to_num(s1 + minv))


def _sc_seg_body(flat_hbm, idx_hbm, perm_hbm, zeros_hbm, seg_hbm, krand_hbm,
                 idxv, rowsv, pidxv, prowv, table_sh, sem):
    c = lax.axis_index("c")
    s = lax.axis_index("s")
    w = s * 2 + c
    pltpu.sync_copy(zeros_hbm, table_sh.at[pl.ds(RPW * s, RPW)])
    plsc.subcore_barrier()
    pltpu.sync_copy(idx_hbm.at[pl.ds(4 * w, 4)], idxv)
    for j in range(4):
        pltpu.sync_copy(flat_hbm.at[pl.ds(RPW * w + 128 * j, 128)], rowsv)
        pltpu.sync_copy(rowsv, table_sh.at[idxv.at[j]], add=True)
    plsc.subcore_barrier()
    pltpu.sync_copy(table_sh.at[pl.ds(RPW * s, RPW)],
                    seg_hbm.at[c, pl.ds(RPW * s, RPW)])
    pltpu.sync_copy(perm_hbm.at[pl.ds(2 * w, 2)], pidxv)
    for j in range(2):
        pltpu.async_copy(flat_hbm.at[pidxv.at[j]], prowv, sem).wait()
        pltpu.sync_copy(prowv, krand_hbm.at[pl.ds(PPW * w + 128 * j, 128)])


_sc_seg = functools.partial(
    pl.kernel,
    mesh=plsc.VectorSubcoreMesh(core_axis_name="c", subcore_axis_name="s"),
    out_type=[
        jax.ShapeDtypeStruct((2, N_EMB, AUG), jnp.float32),
        jax.ShapeDtypeStruct((N_EMB, AUG), jnp.float32),
    ],
    scratch_types=[
        pltpu.VMEM((4, 128), jnp.int32),
        pltpu.VMEM((128, AUG), jnp.float32),
        pltpu.VMEM((2, 128), jnp.int32),
        pltpu.VMEM((128, AUG), jnp.float32),
        pltpu.VMEM_SHARED((N_EMB, AUG), jnp.float32),
        pltpu.SemaphoreType.DMA,
    ],
)(_sc_seg_body)


def _k23(idx_ref, x_ref, seg_ref, emb_ref, krand_ref,
         q_ref, sc_ref, used_ref, comm_ref, newk_scr):
    i = pl.program_id(0)

    @pl.when(i == 0)
    def _():
        seg = seg_ref[0] + seg_ref[1]
        ksum_new = seg[:, :EMB_DIM]
        kelem_new = seg[:, EMB_DIM:EMB_DIM + 1]
        emb = emb_ref[...]
        k_sum = BETA * emb + (1.0 - BETA) * ksum_new
        k_elem = BETA * 1.0 + (1.0 - BETA) * kelem_new
        usage = (k_elem >= THRESHOLD).astype(jnp.float32)
        new_k = (usage * (k_sum / k_elem)
                 + (1.0 - usage) * krand_ref[:, :EMB_DIM])
        newk_scr[...] = new_k
        prob = kelem_new / jnp.sum(kelem_new)
        sc_ref[0] = -jnp.sum(prob * jnp.log(prob + 1e-8))
        sc_ref[1] = jnp.sum(usage)
        diff = new_k - emb
        sc_ref[2] = jnp.sum(diff * diff)
        used_ref[0] = jnp.sum((kelem_new >= THRESHOLD).astype(jnp.int32))
        comm_ref[0] = 0.0

    idx = idx_ref[0, 0, :]
    rows = x_ref[...]
    newk = newk_scr[...]
    codes = jax.lax.broadcasted_iota(jnp.int32, (T, N_EMB), 1)
    one_hot = (codes == idx[:, None]).astype(jnp.float32)
    q = jnp.dot(one_hot, newk, preferred_element_type=jnp.float32)
    q_ref[...] = q
    d = q - rows
    comm_ref[0] += jnp.sum(d * d)


def kernel(x, embeddings):
    xt = jnp.swapaxes(x, 1, -1)
    flat_x = xt.reshape(ROWS, EMB_DIM)
    xaug = jnp.pad(flat_x, ((0, 0), (0, AUG - EMB_DIM)))
    xaug = xaug.at[:, EMB_DIM].set(1.0)
    embT = embeddings.T
    s2 = jnp.sum(embT * embT, axis=0, keepdims=True)
    embM = jnp.concatenate(
        [-2.0 * embT, s2,
         jnp.zeros((AUG - EMB_DIM - 1, N_EMB), jnp.float32)], axis=0)

    idx3, stats = pl.pallas_call(
        _k1,
        grid=(NT,),
        in_specs=[
            pl.BlockSpec((T, AUG), lambda i: (i, 0)),
            pl.BlockSpec((AUG, N_EMB), lambda i: (0, 0)),
        ],
        out_specs=[
            pl.BlockSpec((1, 1, T), lambda i: (i, 0, 0)),
            pl.BlockSpec(memory_space=pltpu.SMEM),
        ],
        out_shape=[
            jax.ShapeDtypeStruct((NT, 1, T), jnp.int32),
            jax.ShapeDtypeStruct((4,), jnp.float32),
        ],
    )(xaug, embM)

    idx2d = idx3.reshape(128, 128)
    perm2d = jnp.asarray(_perm_idx(), jnp.int32).reshape(64, 128)
    zeros = jnp.zeros((RPW, AUG), jnp.float32)
    seg2, krand = _sc_seg(xaug, idx2d, perm2d, zeros)

    q_flat, sc, usedc, comm = pl.pallas_call(
        _k23,
        grid=(NT,),
        in_specs=[
            pl.BlockSpec((1, 1, T), lambda i: (i, 0, 0)),
            pl.BlockSpec((T, EMB_DIM), lambda i: (i, 0)),
            pl.BlockSpec((2, N_EMB, AUG), lambda i: (0, 0, 0)),
            pl.BlockSpec((N_EMB, EMB_DIM), lambda i: (0, 0)),
            pl.BlockSpec((N_EMB, AUG), lambda i: (0, 0)),
        ],
        out_specs=[
            pl.BlockSpec((T, EMB_DIM), lambda i: (i, 0)),
            pl.BlockSpec(memory_space=pltpu.SMEM),
            pl.BlockSpec(memory_space=pltpu.SMEM),
            pl.BlockSpec(memory_space=pltpu.SMEM),
        ],
        out_shape=[
            jax.ShapeDtypeStruct((ROWS, EMB_DIM), jnp.float32),
            jax.ShapeDtypeStruct((4,), jnp.float32),
            jax.ShapeDtypeStruct((1,), jnp.int32),
            jax.ShapeDtypeStruct((1,), jnp.float32),
        ],
        scratch_shapes=[pltpu.VMEM((N_EMB, EMB_DIM), jnp.float32)],
    )(idx3, flat_x, seg2, embeddings, krand)

    quantized = jnp.swapaxes(q_flat.reshape(xt.shape), 1, -1)
    out = quantized

    n = float(ROWS * EMB_DIM)
    mean = stats[0] / n
    prenorm = jnp.sqrt(jnp.maximum(stats[1] - n * mean * mean, 0.0) / n)
    fit = stats[2] / float(ROWS)
    loss = BETA * comm[0] / n
    entropy = sc[0]
    usage_sum = sc[1]
    dk = jnp.nan_to_num(jnp.sqrt(sc[2]) / np.sqrt(float(N_EMB * EMB_DIM)))
    used_curr = usedc[0]
    return (out, quantized, loss, fit, prenorm, entropy, used_curr,
            usage_sum, dk)
